# baseline (speedup 1.0000x reference)
import functools
import math

import jax
import jax.numpy as jnp
from jax import lax
from jax.experimental import pallas as pl
from jax.experimental.pallas import tpu as pltpu
from jax.experimental.pallas import tpu_sc as plsc

_DIM = 256
_T_MAX = 240
_L = 50
_LPAD = 56


def _sin_table():
    t = jnp.arange(_T_MAX, dtype=jnp.float32)[:, None]
    denominator = jnp.exp(
        jnp.arange(_DIM, dtype=jnp.float32) * math.log(10000.0) / _DIM
    )
    base = t / denominator
    col = jnp.arange(_DIM)
    return jnp.where((col % 2) == 0, jnp.sin(base), jnp.cos(base))


def _proj_body(base_ref, w_ref, b_ref, out_ref):
    out_ref[...] = (
        lax.dot_general(
            base_ref[...],
            w_ref[...],
            dimension_numbers=(((1,), (1,)), ((), ())),
            preferred_element_type=jnp.float32,
        )
        + b_ref[...]
    )


_project_table = pl.pallas_call(
    _proj_body,
    out_shape=jax.ShapeDtypeStruct((_T_MAX, _DIM), jnp.float32),
)

_NC = 2
_NS = 16
_NW = _NC * _NS
_BPC = 2
_NBUF = 4


@functools.lru_cache(maxsize=None)
def _make_gather(n_batch):
    assert n_batch % _NW == 0
    b_per_w = n_batch // _NW
    assert b_per_w % (_BPC * _NBUF) == 0
    idx_per_w = b_per_w * _LPAD
    n_chunk = b_per_w // _BPC
    mesh = plsc.VectorSubcoreMesh(
        core_axis_name="c", subcore_axis_name="s", num_cores=_NC, num_subcores=_NS
    )

    @functools.partial(
        pl.kernel,
        out_type=jax.ShapeDtypeStruct((n_batch, _L, _DIM), jnp.float32),
        mesh=mesh,
        compiler_params=pltpu.CompilerParams(use_tc_tiling_on_sc=False),
        scratch_types=[
            pltpu.VMEM((idx_per_w,), jnp.int32),
            pltpu.VMEM((_NBUF, _BPC * _LPAD, _DIM), jnp.float32),
        ]
        + [pltpu.SemaphoreType.DMA] * (2 * _NBUF),
    )
    def gather(proj_hbm, idx_hbm, out_hbm, idx_v, rows_v, *sems):
        gsem = sems[:_NBUF]
        osem = sems[_NBUF:]
        wid = lax.axis_index("s") * _NC + lax.axis_index("c")
        b0 = wid * b_per_w
        pltpu.sync_copy(idx_hbm.at[pl.ds(wid * idx_per_w, idx_per_w)], idx_v)

        def start_gather(g, p):
            pltpu.async_copy(
                proj_hbm.at[idx_v.at[pl.ds(g * (_BPC * _LPAD), _BPC * _LPAD)]],
                rows_v.at[p],
                gsem[p],
            )

        def wait_gather(p):
            pltpu.make_async_copy(
                proj_hbm.at[pl.ds(0, _BPC * _LPAD)], rows_v.at[p], gsem[p]
            ).wait()

        def start_writes(g, p):
            for j in range(_BPC):
                pltpu.async_copy(
                    rows_v.at[p, pl.ds(j * _LPAD, _L)],
                    out_hbm.at[b0 + g * _BPC + j],
                    osem[p],
                )

        def wait_writes(p):
            for j in range(_BPC):
                pltpu.make_async_copy(
                    rows_v.at[p, pl.ds(j * _LPAD, _L)],
                    out_hbm.at[0],
                    osem[p],
                ).wait()

        for p in range(_NBUF):
            start_gather(p, p)

        def body(i, carry):
            for p in range(_NBUF):
                g = i * _NBUF + p
                wait_gather(p)
                start_writes(g, p)
                wait_writes(p)

                @pl.when(g + _NBUF < n_chunk)
                def _():
                    start_gather(g + _NBUF, p)

            return carry

        lax.fori_loop(0, n_chunk // _NBUF, body, 0)

    return gather


def kernel(delta_t, W, b):
    base = _sin_table()
    proj = _project_table(base, W, b.reshape(1, _DIM))
    idx = delta_t.astype(jnp.int32)
    idx = jnp.pad(idx, ((0, 0), (0, _LPAD - _L))).reshape(-1)
    out = _make_gather(delta_t.shape[0])(proj, idx)
    return out


# speedup vs baseline: 3.5607x; 3.5607x over previous
import functools
import math

import jax
import jax.numpy as jnp
from jax import lax
from jax.experimental import pallas as pl
from jax.experimental.pallas import tpu as pltpu
from jax.experimental.pallas import tpu_sc as plsc

_DIM = 256
_T_MAX = 240
_L = 50
_LPAD = 56


def _sin_table():
    t = jnp.arange(_T_MAX, dtype=jnp.float32)[:, None]
    denominator = jnp.exp(
        jnp.arange(_DIM, dtype=jnp.float32) * math.log(10000.0) / _DIM
    )
    base = t / denominator
    col = jnp.arange(_DIM)
    return jnp.where((col % 2) == 0, jnp.sin(base), jnp.cos(base))


def _proj_body(base_ref, w_ref, b_ref, out_ref):
    out_ref[...] = (
        lax.dot_general(
            base_ref[...],
            w_ref[...],
            dimension_numbers=(((1,), (1,)), ((), ())),
            preferred_element_type=jnp.float32,
        )
        + b_ref[...]
    )


_project_table = pl.pallas_call(
    _proj_body,
    out_shape=jax.ShapeDtypeStruct((_T_MAX, _DIM), jnp.float32),
)

_NC = 2
_NS = 16
_NW = _NC * _NS
_NBUF = 8


@functools.lru_cache(maxsize=None)
def _make_gather(n_batch):
    assert n_batch % _NW == 0
    b_per_w = n_batch // _NW
    assert b_per_w % _NBUF == 0
    idx_per_w = b_per_w * _LPAD
    mesh = plsc.VectorSubcoreMesh(
        core_axis_name="c", subcore_axis_name="s", num_cores=_NC, num_subcores=_NS
    )

    @functools.partial(
        pl.kernel,
        out_type=jax.ShapeDtypeStruct((n_batch, _L, _DIM), jnp.float32),
        mesh=mesh,
        scratch_types=[
            pltpu.VMEM((idx_per_w,), jnp.int32),
            pltpu.VMEM((_NBUF, _L, _DIM), jnp.float32),
        ]
        + [pltpu.SemaphoreType.DMA] * (2 * _NBUF),
    )
    def gather(proj_hbm, idx_hbm, out_hbm, idx_v, rows_v, *sems):
        gsem = sems[:_NBUF]
        osem = sems[_NBUF:]
        wid = lax.axis_index("s") * _NC + lax.axis_index("c")
        b0 = wid * b_per_w
        pltpu.sync_copy(idx_hbm.at[pl.ds(wid * idx_per_w, idx_per_w)], idx_v)

        def gather_desc(bb, p):
            return pltpu.make_async_copy(
                proj_hbm.at[idx_v.at[pl.ds(bb * _LPAD, _L)]],
                rows_v.at[p],
                gsem[p],
            )

        def start_write(bb, p):
            pltpu.async_copy(rows_v.at[p], out_hbm.at[b0 + bb], osem[p])

        def wait_write(p):
            pltpu.make_async_copy(rows_v.at[p], out_hbm.at[0], osem[p]).wait()

        for p in range(_NBUF):
            gather_desc(p, p).start()

        def body(i, carry):
            for p in range(_NBUF):
                bb = i * _NBUF + p
                gather_desc(bb, p).wait()
                start_write(bb, p)
                wait_write(p)

                @pl.when(bb + _NBUF < b_per_w)
                def _():
                    gather_desc(bb + _NBUF, p).start()

            return carry

        lax.fori_loop(0, b_per_w // _NBUF, body, 0)

    return gather


def kernel(delta_t, W, b):
    base = _sin_table()
    proj = _project_table(base, W, b.reshape(1, _DIM))
    idx = delta_t.astype(jnp.int32)
    idx = jnp.pad(idx, ((0, 0), (0, _LPAD - _L))).reshape(-1)
    out = _make_gather(delta_t.shape[0])(proj, idx)
    return out


# speedup vs baseline: 5.4464x; 1.5296x over previous
import functools
import math

import jax
import jax.numpy as jnp
from jax import lax
from jax.experimental import pallas as pl
from jax.experimental.pallas import tpu as pltpu
from jax.experimental.pallas import tpu_sc as plsc

_DIM = 256
_T_MAX = 240


def _sin_table():
    t = jnp.arange(_T_MAX, dtype=jnp.float32)[:, None]
    denominator = jnp.exp(
        jnp.arange(_DIM, dtype=jnp.float32) * math.log(10000.0) / _DIM
    )
    base = t / denominator
    col = jnp.arange(_DIM)
    return jnp.where((col % 2) == 0, jnp.sin(base), jnp.cos(base))


def _proj_body(base_ref, w_ref, b_ref, out_ref):
    out_ref[...] = (
        lax.dot_general(
            base_ref[...],
            w_ref[...],
            dimension_numbers=(((1,), (1,)), ((), ())),
            preferred_element_type=jnp.float32,
        )
        + b_ref[...]
    )


_project_table = pl.pallas_call(
    _proj_body,
    out_shape=jax.ShapeDtypeStruct((_T_MAX, _DIM), jnp.float32),
)

_NC = 2
_NS = 16
_NW = _NC * _NS
_CHUNK = 128
_NBUF = 2


@functools.lru_cache(maxsize=None)
def _make_gather(n_rows):
    assert n_rows % (_NW * _CHUNK) == 0
    per_w = n_rows // _NW
    n_chunk = per_w // _CHUNK
    assert n_chunk % _NBUF == 0
    mesh = plsc.VectorSubcoreMesh(
        core_axis_name="c", subcore_axis_name="s", num_cores=_NC, num_subcores=_NS
    )

    @functools.partial(
        pl.kernel,
        out_type=jax.ShapeDtypeStruct((n_rows, _DIM), jnp.float32),
        mesh=mesh,
        scratch_types=[
            pltpu.VMEM((per_w,), jnp.int32),
            pltpu.VMEM((_NBUF, _CHUNK, _DIM), jnp.float32),
        ]
        + [pltpu.SemaphoreType.DMA] * (2 * _NBUF),
    )
    def gather(proj_hbm, idx_hbm, out_hbm, idx_v, rows_v, *sems):
        gsem = sems[:_NBUF]
        osem = sems[_NBUF:]
        wid = lax.axis_index("s") * _NC + lax.axis_index("c")
        row0 = wid * per_w
        pltpu.sync_copy(idx_hbm.at[pl.ds(row0, per_w)], idx_v)

        def start_gather(g, p):
            pltpu.async_copy(
                proj_hbm.at[idx_v.at[pl.ds(g * _CHUNK, _CHUNK)]],
                rows_v.at[p],
                gsem[p],
            )

        def wait_gather(p):
            pltpu.make_async_copy(
                proj_hbm.at[pl.ds(0, _CHUNK)], rows_v.at[p], gsem[p]
            ).wait()

        def start_write(g, p):
            pltpu.async_copy(
                rows_v.at[p],
                out_hbm.at[pl.ds(row0 + g * _CHUNK, _CHUNK)],
                osem[p],
            )

        def wait_write(p):
            pltpu.make_async_copy(
                rows_v.at[p], out_hbm.at[pl.ds(0, _CHUNK)], osem[p]
            ).wait()

        for p in range(_NBUF):
            start_gather(p, p)

        def body(i, carry):
            for p in range(_NBUF):
                g = i * _NBUF + p
                wait_gather(p)
                start_write(g, p)
                wait_write(p)

                @pl.when(g + _NBUF < n_chunk)
                def _():
                    start_gather(g + _NBUF, p)

            return carry

        lax.fori_loop(0, n_chunk // _NBUF, body, 0)

    return gather


def kernel(delta_t, W, b):
    B, L = delta_t.shape
    base = _sin_table()
    proj = _project_table(base, W, b.reshape(1, _DIM))
    idx = delta_t.T.reshape(-1).astype(jnp.int32)
    out = _make_gather(idx.shape[0])(proj, idx)
    return out.reshape(L, B, _DIM).transpose(1, 0, 2)
